# baseline (speedup 1.0000x reference)
import jax
import jax.numpy as jnp
from jax import lax
from jax.experimental import pallas as pl
from jax.experimental.pallas import tpu as pltpu

_N = 5000
_ROWS = 40
_L = 128
_PAD = _ROWS * _L
_IOU_THRESH = 0.6
_MIN_SIZE = 5.0
_MAX_SIZE = 300.0
_MIN_ASPECT = 0.5
_MAX_ASPECT = 8.0
_MIN_CONFIDENCE = 0.3


def _nms_filter_kernel(dets_ref, out_ref, act_ref, area_ref):
    f32 = jnp.float32
    ones_mat = jnp.ones((_L, _L), f32)
    sub_i = lax.broadcasted_iota(jnp.int32, (_L, _L), 0)
    lane_i = lax.broadcasted_iota(jnp.int32, (_L, _L), 1)
    eye = (sub_i == lane_i).astype(f32)
    rank_lt = sub_i < lane_i

    x1a = dets_ref[1]
    y1a = dets_ref[2]
    x2a = dets_ref[3]
    y2a = dets_ref[4]
    area_ref[...] = (jnp.maximum(x2a - x1a, 0.0)
                     * jnp.maximum(y2a - y1a, 0.0))
    act_ref[...] = jnp.ones((_ROWS, 1, _L), f32)

    def row_to_colb(v_row):
        v_rb = jnp.broadcast_to(v_row, (_L, _L))
        return v_rb.T

    def outer(r, carry):
        x1r = dets_ref[1, r]
        y1r = dets_ref[2, r]
        x2r = dets_ref[3, r]
        y2r = dets_ref[4, r]
        arear = area_ref[r]
        x1cb = row_to_colb(x1r)
        y1cb = row_to_colb(y1r)
        x2cb = row_to_colb(x2r)
        y2cb = row_to_colb(y2r)
        area_cb = row_to_colb(arear)

        x1rb = jnp.broadcast_to(x1r, (_L, _L))
        y1rb = jnp.broadcast_to(y1r, (_L, _L))
        x2rb = jnp.broadcast_to(x2r, (_L, _L))
        y2rb = jnp.broadcast_to(y2r, (_L, _L))
        area_rb = jnp.broadcast_to(arear, (_L, _L))
        xx1 = jnp.maximum(x1cb, x1rb)
        yy1 = jnp.maximum(y1cb, y1rb)
        xx2 = jnp.minimum(x2cb, x2rb)
        yy2 = jnp.minimum(y2cb, y2rb)
        inter = jnp.maximum(xx2 - xx1, 0.0) * jnp.maximum(yy2 - yy1, 0.0)
        iou = inter / (area_cb + area_rb - inter + 1e-9)
        m_mat = jnp.where((iou > _IOU_THRESH) & rank_lt, 1.0, 0.0)

        act_pre = act_ref[r]

        def cond(st):
            return st[2]

        def fbody(st):
            k_row, kcb, _ = st
            sup = jnp.max(m_mat * kcb, axis=0, keepdims=True) > 0.0
            k_new = jnp.where(sup, 0.0, act_pre)
            changed = jnp.any(k_new != k_row)
            return (k_new, row_to_colb(k_new), changed)

        k_fin, kcb_fin, _ = lax.while_loop(
            cond, fbody, (act_pre, row_to_colb(act_pre), jnp.bool_(True))
        )
        act_ref[r] = k_fin

        def cross(m, c2):
            x1m = jnp.broadcast_to(dets_ref[1, m], (_L, _L))
            y1m = jnp.broadcast_to(dets_ref[2, m], (_L, _L))
            x2m = jnp.broadcast_to(dets_ref[3, m], (_L, _L))
            y2m = jnp.broadcast_to(dets_ref[4, m], (_L, _L))
            aream = jnp.broadcast_to(area_ref[m], (_L, _L))
            cx1 = jnp.maximum(x1cb, x1m)
            cy1 = jnp.maximum(y1cb, y1m)
            cx2 = jnp.minimum(x2cb, x2m)
            cy2 = jnp.minimum(y2cb, y2m)
            cin = jnp.maximum(cx2 - cx1, 0.0) * jnp.maximum(cy2 - cy1, 0.0)
            ciou = cin / (area_cb + aream - cin + 1e-9)
            hit = jnp.where(ciou > _IOU_THRESH, kcb_fin, 0.0)
            sup2 = jnp.max(hit, axis=0, keepdims=True) > 0.0
            act_ref[m] = jnp.where(sup2, 0.0, act_ref[m])
            return c2

        lax.fori_loop(r + 1, _ROWS, cross, 0)
        return carry

    lax.fori_loop(0, _ROWS, outer, 0)

    keep = act_ref[...] > 0.0
    w = x2a - x1a
    h = y2a - y1a
    aspect = jnp.where(w > 0.0, h / jnp.maximum(w, 1e-9), 0.0)
    size_ok = (
        (w >= _MIN_SIZE)
        & (h >= _MIN_SIZE)
        & (w <= _MAX_SIZE)
        & (h <= _MAX_SIZE)
        & (aspect >= _MIN_ASPECT)
        & (aspect <= _MAX_ASPECT)
    )
    s0 = dets_ref[5]
    s1 = dets_ref[6]
    s2 = dets_ref[7]
    s3 = dets_ref[8]
    conf = jnp.maximum(jnp.maximum(s0, s1), jnp.maximum(s2, s3))
    conf_ok = conf >= _MIN_CONFIDENCE
    valid_type = jnp.maximum(jnp.maximum(s1, s2), s3) > s0
    fmask = (keep & size_ok & conf_ok & valid_type).astype(f32)
    for c in range(9):
        out_ref[c] = dets_ref[c] * fmask


def _run_nms(dets9):
    return pl.pallas_call(
        _nms_filter_kernel,
        out_shape=jax.ShapeDtypeStruct((9, _ROWS, 1, _L), jnp.float32),
        scratch_shapes=[
            pltpu.VMEM((_ROWS, 1, _L), jnp.float32),
            pltpu.VMEM((_ROWS, 1, _L), jnp.float32),
        ],
    )(dets9)


def kernel(detections):
    scores = jnp.max(detections[:, 5:9], axis=1)
    order = jnp.argsort(-scores)
    det_s = jnp.take(detections, order, axis=0)
    padded = jnp.zeros((_PAD, 9), jnp.float32).at[:_N].set(det_s)
    dets9 = padded.T.reshape(9, _ROWS, 1, _L)
    out = _run_nms(dets9)
    return out.reshape(9, _PAD).T[:_N]
